# Initial kernel scaffold; baseline (speedup 1.0000x reference)
import jax
import jax.numpy as jnp
from jax.experimental import pallas as pl


def kernel(x, pos_emb, route_W, route_b, W1, b1, W2, b2, return_logits):
    raise NotImplementedError("write your pallas kernel here")


# speedup vs baseline: 6.5073x; 6.5073x over previous
import functools

import jax
import jax.numpy as jnp
from jax import lax
from jax.experimental import pallas as pl
from jax.experimental.pallas import tpu as pltpu
from jax.experimental.pallas import tpu_sc as plsc

DIM = 768
DEPTH = 2
E = 64
FF = 1024
SEQ = 2048

T = 128
NCH = 80
NPAD = 2816
NW = 32


def _make_sc_gather(n_out, d):
    n_per = n_out // NW

    @functools.partial(
        pl.kernel,
        mesh=plsc.VectorSubcoreMesh(core_axis_name="c", subcore_axis_name="s"),
        out_type=jax.ShapeDtypeStruct((n_out, d), jnp.float32),
        scratch_types=[
            pltpu.VMEM((n_per,), jnp.int32),
            pltpu.VMEM((n_per, d), jnp.float32),
            pltpu.SemaphoreType.DMA,
        ],
    )
    def gath(table_hbm, idx_hbm, out_hbm, idx_v, rows_v, sem):
        wid = lax.axis_index("s") * 2 + lax.axis_index("c")
        base = wid * n_per
        pltpu.sync_copy(idx_hbm.at[pl.ds(base, n_per)], idx_v)
        pltpu.async_copy(table_hbm.at[idx_v], rows_v, sem).wait()
        pltpu.sync_copy(rows_v, out_hbm.at[pl.ds(base, n_per)])

    return gath


def _route0_body(x_ref, p_ref, w_ref, b_ref, h_ref, sel_ref):
    h = x_ref[...] + p_ref[...]
    h_ref[...] = h
    logits = jnp.dot(h, w_ref[...], preferred_element_type=jnp.float32)
    logits = logits + b_ref[...]
    m = jnp.max(logits, axis=1, keepdims=True)
    ii = lax.broadcasted_iota(jnp.int32, logits.shape, 1)
    sel_ref[...] = jnp.min(jnp.where(logits >= m, ii, E), axis=1, keepdims=True)


def _route1_body(h_ref, w_ref, b_ref, sel_ref):
    logits = jnp.dot(h_ref[...], w_ref[...], preferred_element_type=jnp.float32)
    logits = logits + b_ref[...]
    m = jnp.max(logits, axis=1, keepdims=True)
    ii = lax.broadcasted_iota(jnp.int32, logits.shape, 1)
    sel_ref[...] = jnp.min(jnp.where(logits >= m, ii, E), axis=1, keepdims=True)


_route0 = pl.pallas_call(
    _route0_body,
    out_shape=(jax.ShapeDtypeStruct((SEQ, DIM), jnp.float32),
               jax.ShapeDtypeStruct((SEQ, 1), jnp.int32)),
)

_route1 = pl.pallas_call(
    _route1_body,
    out_shape=jax.ShapeDtypeStruct((SEQ, 1), jnp.int32),
)


def _ffn_body(ce_ref, cs_ref, hs_ref, w1_ref, b1_ref, w2_ref, b2_ref, os_ref):
    i = pl.program_id(0)
    e = ce_ref[i]
    s = pl.multiple_of(cs_ref[i], 8)
    x = hs_ref[pl.ds(s, T), :]
    h = jnp.dot(x, w1_ref[0, 0], preferred_element_type=jnp.float32)
    h = h + b1_ref[0, pl.ds(e, 1), :]
    h = 0.5 * h * (1.0 + lax.erf(h * 0.7071067811865476))
    y = jnp.dot(h, w2_ref[0, 0], preferred_element_type=jnp.float32)
    y = y + b2_ref[0, pl.ds(e, 1), :]
    os_ref[pl.ds(s, T), :] = y + x


def _make_ffn(l):
    return pl.pallas_call(
        _ffn_body,
        grid_spec=pltpu.PrefetchScalarGridSpec(
            num_scalar_prefetch=2,
            grid=(NCH,),
            in_specs=[
                pl.BlockSpec((NPAD, DIM), lambda i, ce, cs: (0, 0)),
                pl.BlockSpec((1, 1, DIM, FF), lambda i, ce, cs: (l, ce[i], 0, 0)),
                pl.BlockSpec((1, E, FF), lambda i, ce, cs: (l, 0, 0)),
                pl.BlockSpec((1, 1, FF, DIM), lambda i, ce, cs: (l, ce[i], 0, 0)),
                pl.BlockSpec((1, E, DIM), lambda i, ce, cs: (l, 0, 0)),
            ],
            out_specs=pl.BlockSpec((NPAD, DIM), lambda i, ce, cs: (0, 0)),
        ),
        out_shape=jax.ShapeDtypeStruct((NPAD, DIM), jnp.float32),
    )


def _dispatch(sel):
    onehot = (sel[:, None] == jnp.arange(E, dtype=jnp.int32)[None, :])
    csum = jnp.cumsum(onehot.astype(jnp.int32), axis=0)
    counts = csum[-1]
    rank = jnp.take_along_axis(csum, sel[:, None], axis=1)[:, 0] - 1
    cap = ((counts + 7) // 8) * 8
    start = jnp.cumsum(cap) - cap
    slot = start[sel] + rank
    tok_at_slot = jnp.zeros((NPAD,), jnp.int32).at[slot].set(
        jnp.arange(SEQ, dtype=jnp.int32))
    nch = jnp.where(counts > 0, (cap + T - 1) // T, 0)
    cum_nch = jnp.cumsum(nch)
    excl_nch = cum_nch - nch
    total = cum_nch[-1]
    ii = jnp.arange(NCH, dtype=jnp.int32)
    e_arr = jnp.minimum(
        jnp.searchsorted(cum_nch, ii, side="right"), E - 1).astype(jnp.int32)
    valid = ii < total
    e_last = jnp.max(jnp.where(valid, e_arr, -1)).astype(jnp.int32)
    ce = jnp.where(valid, e_arr, e_last)
    cs = jnp.where(valid, start[e_arr] + (ii - excl_nch[e_arr]) * T,
                   NPAD - T).astype(jnp.int32)
    return slot, tok_at_slot, ce, cs


def kernel(x, pos_emb, route_W, route_b, W1, b1, W2, b2, return_logits):
    x2 = x.reshape(SEQ, DIM)
    p2 = pos_emb.reshape(-1, DIM)[:SEQ]
    gather_np = _make_sc_gather(NPAD, DIM)
    gather_sq = _make_sc_gather(SEQ, DIM)
    h = None
    for l in range(DEPTH):
        if l == 0:
            h, sel2 = _route0(x2, p2, route_W[0], route_b[0].reshape(1, E))
        else:
            sel2 = _route1(h, route_W[1], route_b[1].reshape(1, E))
        slot, tok_at_slot, ce, cs = _dispatch(sel2[:, 0])
        h_sorted = gather_np(h, tok_at_slot)
        out_sorted = _make_ffn(l)(ce, cs, h_sorted, W1, b1, W2, b2)
        h = gather_sq(out_sorted, slot)
    return h.reshape(1, SEQ, DIM)


# speedup vs baseline: 10.6925x; 1.6432x over previous
import functools

import jax
import jax.numpy as jnp
from jax import lax
from jax.experimental import pallas as pl
from jax.experimental.pallas import tpu as pltpu
from jax.experimental.pallas import tpu_sc as plsc

DIM = 768
DEPTH = 2
E = 64
FF = 1024
SEQ = 2048

T = 128
NCH = 80
NPAD = 2816
NW = 32


def _make_sc_scatter(d):
    n_per = SEQ // NW

    @functools.partial(
        pl.kernel,
        mesh=plsc.VectorSubcoreMesh(core_axis_name="c", subcore_axis_name="s"),
        out_type=jax.ShapeDtypeStruct((NPAD, d), jnp.float32),
        scratch_types=[
            pltpu.VMEM((n_per,), jnp.int32),
            pltpu.VMEM((n_per, d), jnp.float32),
            pltpu.SemaphoreType.DMA,
        ],
    )
    def scat(src_hbm, idx_hbm, out_hbm, idx_v, rows_v, sem):
        wid = lax.axis_index("s") * 2 + lax.axis_index("c")
        base = wid * n_per
        pltpu.sync_copy(idx_hbm.at[wid], idx_v)
        pltpu.sync_copy(src_hbm.at[pl.ds(base, n_per)], rows_v)
        pltpu.async_copy(rows_v, out_hbm.at[idx_v], sem).wait()

    return scat


def _make_sc_gather(n_out, d):
    n_per = n_out // NW

    @functools.partial(
        pl.kernel,
        mesh=plsc.VectorSubcoreMesh(core_axis_name="c", subcore_axis_name="s"),
        out_type=jax.ShapeDtypeStruct((n_out, d), jnp.float32),
        scratch_types=[
            pltpu.VMEM((n_per,), jnp.int32),
            pltpu.VMEM((n_per, d), jnp.float32),
            pltpu.SemaphoreType.DMA,
        ],
    )
    def gath(table_hbm, idx_hbm, out_hbm, idx_v, rows_v, sem):
        wid = lax.axis_index("s") * 2 + lax.axis_index("c")
        base = wid * n_per
        pltpu.sync_copy(idx_hbm.at[pl.ds(base, n_per)], idx_v)
        pltpu.async_copy(table_hbm.at[idx_v], rows_v, sem).wait()
        pltpu.sync_copy(rows_v, out_hbm.at[pl.ds(base, n_per)])

    return gath


def _route_dispatch(h):
    logits = h
    m = jnp.max(logits, axis=1, keepdims=True)
    lane = lax.broadcasted_iota(jnp.int32, (SEQ, E), 1)
    sel = jnp.min(jnp.where(logits >= m, lane, E), axis=1, keepdims=True)
    onehot = (lane == sel).astype(jnp.float32)
    counts = jnp.sum(onehot, axis=0, keepdims=True)
    r = lax.broadcasted_iota(jnp.int32, (SEQ, SEQ), 0)
    c = lax.broadcasted_iota(jnp.int32, (SEQ, SEQ), 1)
    tril = (c < r).astype(jnp.float32)
    before = jnp.dot(tril, onehot, preferred_element_type=jnp.float32)
    rank = jnp.sum(before * onehot, axis=1, keepdims=True)
    cap = (((counts.astype(jnp.int32) + 7) // 8) * 8).astype(jnp.float32)
    er = lax.broadcasted_iota(jnp.int32, (E, E), 0)
    ec = lax.broadcasted_iota(jnp.int32, (E, E), 1)
    trile = (er < ec).astype(jnp.float32)
    start = jax.lax.dot_general(
        cap, trile, (((1,), (0,)), ((), ())),
        precision=jax.lax.Precision.HIGHEST,
        preferred_element_type=jnp.float32)
    slot = jnp.sum(onehot * start, axis=1, keepdims=True) + rank
    return slot.astype(jnp.int32), counts.astype(jnp.int32)


def _route0_body(x_ref, p_ref, w_ref, b_ref, h_ref, slot_ref, cnt_ref):
    h = x_ref[...] + p_ref[...]
    h_ref[...] = h
    logits = jnp.dot(h, w_ref[...], preferred_element_type=jnp.float32)
    slot_ref[...], cnt_ref[...] = _route_dispatch(logits + b_ref[...])


def _route1_body(h_ref, w_ref, b_ref, slot_ref, cnt_ref):
    logits = jnp.dot(h_ref[...], w_ref[...], preferred_element_type=jnp.float32)
    slot_ref[...], cnt_ref[...] = _route_dispatch(logits + b_ref[...])


_route0 = pl.pallas_call(
    _route0_body,
    out_shape=(jax.ShapeDtypeStruct((SEQ, DIM), jnp.float32),
               jax.ShapeDtypeStruct((SEQ, 1), jnp.int32),
               jax.ShapeDtypeStruct((1, E), jnp.int32)),
)

_route1 = pl.pallas_call(
    _route1_body,
    out_shape=(jax.ShapeDtypeStruct((SEQ, 1), jnp.int32),
               jax.ShapeDtypeStruct((1, E), jnp.int32)),
)


def _ffn_body(ce_ref, cs_ref, hs_ref, w1_ref, b1_ref, w2_ref, b2_ref, os_ref):
    i = pl.program_id(0)
    e = ce_ref[i]
    s = pl.multiple_of(cs_ref[i], 8)
    x = hs_ref[pl.ds(s, T), :]
    h = jnp.dot(x, w1_ref[0, 0], preferred_element_type=jnp.float32)
    h = h + b1_ref[0, pl.ds(e, 1), :]
    h = 0.5 * h * (1.0 + lax.erf(h * 0.7071067811865476))
    y = jnp.dot(h, w2_ref[0, 0], preferred_element_type=jnp.float32)
    y = y + b2_ref[0, pl.ds(e, 1), :]
    os_ref[pl.ds(s, T), :] = y + x


def _make_ffn(l):
    return pl.pallas_call(
        _ffn_body,
        grid_spec=pltpu.PrefetchScalarGridSpec(
            num_scalar_prefetch=2,
            grid=(NCH,),
            in_specs=[
                pl.BlockSpec((NPAD, DIM), lambda i, ce, cs: (0, 0)),
                pl.BlockSpec((1, 1, DIM, FF), lambda i, ce, cs: (l, ce[i], 0, 0)),
                pl.BlockSpec((1, E, FF), lambda i, ce, cs: (l, 0, 0)),
                pl.BlockSpec((1, 1, FF, DIM), lambda i, ce, cs: (l, ce[i], 0, 0)),
                pl.BlockSpec((1, E, DIM), lambda i, ce, cs: (l, 0, 0)),
            ],
            out_specs=pl.BlockSpec((NPAD, DIM), lambda i, ce, cs: (0, 0)),
        ),
        out_shape=jax.ShapeDtypeStruct((NPAD, DIM), jnp.float32),
    )


def _chunks(counts):
    cap = ((counts + 7) // 8) * 8
    start = jnp.cumsum(cap) - cap
    nch = jnp.where(counts > 0, (cap + T - 1) // T, 0)
    cum = jnp.cumsum(nch)
    excl = cum - nch
    total = cum[-1]
    ii = jnp.arange(NCH, dtype=jnp.int32)
    e_arr = jnp.minimum(
        jnp.searchsorted(cum, ii, side="right"), E - 1).astype(jnp.int32)
    valid = ii < total
    e_last = jnp.max(jnp.where(valid, e_arr, -1)).astype(jnp.int32)
    ce = jnp.where(valid, e_arr, e_last)
    cs = jnp.where(valid, start[e_arr] + (ii - excl[e_arr]) * T,
                   NPAD - T).astype(jnp.int32)
    return ce, cs


def kernel(x, pos_emb, route_W, route_b, W1, b1, W2, b2, return_logits):
    x2 = x.reshape(SEQ, DIM)
    p2 = pos_emb.reshape(-1, DIM)[:SEQ]
    scatter_np = _make_sc_scatter(DIM)
    gather_sq = _make_sc_gather(SEQ, DIM)

    h0, slot0, cnt0 = _route0(x2, p2, route_W[0], route_b[0].reshape(1, E))
    ce0, cs0 = _chunks(cnt0[0])
    hs0 = scatter_np(h0, slot0.reshape(NW, SEQ // NW))
    os0 = _make_ffn(0)(ce0, cs0, hs0, W1, b1, W2, b2)
    h1 = gather_sq(os0, slot0.reshape(SEQ))

    slot1, cnt1 = _route1(h1, route_W[1], route_b[1].reshape(1, E))
    ce1, cs1 = _chunks(cnt1[0])
    hs1 = scatter_np(h1, slot1.reshape(NW, SEQ // NW))
    os1 = _make_ffn(1)(ce1, cs1, hs1, W1, b1, W2, b2)
    h2 = gather_sq(os1, slot1.reshape(SEQ))
    return h2.reshape(1, SEQ, DIM)


# speedup vs baseline: 11.1491x; 1.0427x over previous
import functools

import jax
import jax.numpy as jnp
from jax import lax
from jax.experimental import pallas as pl
from jax.experimental.pallas import tpu as pltpu
from jax.experimental.pallas import tpu_sc as plsc

DIM = 768
DEPTH = 2
E = 64
FF = 1024
SEQ = 2048

T = 128
NCH = 80
NCHP = 128
NPAD = 2816
NW = 32


def _make_sc_scatter(d):
    n_per = SEQ // NW

    @functools.partial(
        pl.kernel,
        mesh=plsc.VectorSubcoreMesh(core_axis_name="c", subcore_axis_name="s"),
        out_type=jax.ShapeDtypeStruct((NPAD, d), jnp.float32),
        scratch_types=[
            pltpu.VMEM((n_per,), jnp.int32),
            pltpu.VMEM((n_per, d), jnp.float32),
            pltpu.SemaphoreType.DMA,
        ],
    )
    def scat(src_hbm, idx_hbm, out_hbm, idx_v, rows_v, sem):
        wid = lax.axis_index("s") * 2 + lax.axis_index("c")
        base = wid * n_per
        pltpu.sync_copy(idx_hbm.at[wid], idx_v)
        pltpu.sync_copy(src_hbm.at[pl.ds(base, n_per)], rows_v)
        pltpu.async_copy(rows_v, out_hbm.at[idx_v], sem).wait()

    return scat


def _make_sc_gather(n_out, d):
    n_per = n_out // NW

    @functools.partial(
        pl.kernel,
        mesh=plsc.VectorSubcoreMesh(core_axis_name="c", subcore_axis_name="s"),
        out_type=jax.ShapeDtypeStruct((n_out, d), jnp.float32),
        scratch_types=[
            pltpu.VMEM((n_per,), jnp.int32),
            pltpu.VMEM((n_per, d), jnp.float32),
            pltpu.SemaphoreType.DMA,
        ],
    )
    def gath(table_hbm, idx_hbm, out_hbm, idx_v, rows_v, sem):
        wid = lax.axis_index("s") * 2 + lax.axis_index("c")
        base = wid * n_per
        pltpu.sync_copy(idx_hbm.at[pl.ds(base, n_per)], idx_v)
        pltpu.async_copy(table_hbm.at[idx_v], rows_v, sem).wait()
        pltpu.sync_copy(rows_v, out_hbm.at[pl.ds(base, n_per)])

    return gath


def _route_dispatch(h):
    logits = h
    m = jnp.max(logits, axis=1, keepdims=True)
    lane = lax.broadcasted_iota(jnp.int32, (SEQ, E), 1)
    sel = jnp.min(jnp.where(logits >= m, lane, E), axis=1, keepdims=True)
    onehot = (lane == sel).astype(jnp.float32)
    counts = jnp.sum(onehot, axis=0, keepdims=True)
    r = lax.broadcasted_iota(jnp.int32, (SEQ, SEQ), 0)
    c = lax.broadcasted_iota(jnp.int32, (SEQ, SEQ), 1)
    tril = (c < r).astype(jnp.float32)
    before = jnp.dot(tril, onehot, preferred_element_type=jnp.float32)
    rank = jnp.sum(before * onehot, axis=1, keepdims=True)
    cap = (((counts.astype(jnp.int32) + 7) // 8) * 8).astype(jnp.float32)
    er = lax.broadcasted_iota(jnp.int32, (E, E), 0)
    ec = lax.broadcasted_iota(jnp.int32, (E, E), 1)
    trile = (er < ec).astype(jnp.float32)
    start = jax.lax.dot_general(
        cap, trile, (((1,), (0,)), ((), ())),
        precision=jax.lax.Precision.HIGHEST,
        preferred_element_type=jnp.float32)
    slot = jnp.sum(onehot * start, axis=1, keepdims=True) + rank

    counts_i = counts.astype(jnp.int32)
    capi = cap.astype(jnp.int32)
    nch = jnp.where(counts_i > 0, (capi + T - 1) // T, 0)
    trili = (er <= ec).astype(jnp.float32)
    cum = jax.lax.dot_general(
        nch.astype(jnp.float32), trili, (((1,), (0,)), ((), ())),
        precision=jax.lax.Precision.HIGHEST,
        preferred_element_type=jnp.float32)
    excl = cum - nch.astype(jnp.float32)
    total = jnp.max(cum)
    ii = lax.broadcasted_iota(jnp.int32, (NCHP, E), 0)
    e_arr = jnp.minimum(
        jnp.sum((jnp.broadcast_to(cum, (NCHP, E)) <= ii.astype(jnp.float32))
                .astype(jnp.int32), axis=1, keepdims=True), E - 1)
    lane_c = lax.broadcasted_iota(jnp.int32, (NCHP, E), 1)
    oh_e = (lane_c == e_arr).astype(jnp.float32)
    start_g = jnp.sum(oh_e * start, axis=1, keepdims=True)
    excl_g = jnp.sum(oh_e * excl, axis=1, keepdims=True)
    ii_col = lax.broadcasted_iota(jnp.int32, (NCHP, 1), 0)
    valid = ii_col.astype(jnp.float32) < total
    e_last = jnp.max(jnp.where(valid, e_arr, -1))
    ce = jnp.where(valid, e_arr, e_last)
    cs = jnp.where(valid,
                   (start_g + (ii_col.astype(jnp.float32) - excl_g) * T)
                   .astype(jnp.int32),
                   NPAD - T)
    return slot.astype(jnp.int32), ce, cs


def _route0_body(x_ref, p_ref, w_ref, b_ref, h_ref, slot_ref, ce_ref, cs_ref):
    h = x_ref[...] + p_ref[...]
    h_ref[...] = h
    logits = jnp.dot(h, w_ref[...], preferred_element_type=jnp.float32)
    slot_ref[...], ce_ref[...], cs_ref[...] = _route_dispatch(
        logits + b_ref[...])


def _route1_body(h_ref, w_ref, b_ref, slot_ref, ce_ref, cs_ref):
    logits = jnp.dot(h_ref[...], w_ref[...], preferred_element_type=jnp.float32)
    slot_ref[...], ce_ref[...], cs_ref[...] = _route_dispatch(
        logits + b_ref[...])


_route0 = pl.pallas_call(
    _route0_body,
    out_shape=(jax.ShapeDtypeStruct((SEQ, DIM), jnp.float32),
               jax.ShapeDtypeStruct((SEQ, 1), jnp.int32),
               jax.ShapeDtypeStruct((NCHP, 1), jnp.int32),
               jax.ShapeDtypeStruct((NCHP, 1), jnp.int32)),
)

_route1 = pl.pallas_call(
    _route1_body,
    out_shape=(jax.ShapeDtypeStruct((SEQ, 1), jnp.int32),
               jax.ShapeDtypeStruct((NCHP, 1), jnp.int32),
               jax.ShapeDtypeStruct((NCHP, 1), jnp.int32)),
)


def _ffn_body(ce_ref, cs_ref, hs_ref, w1_ref, b1_ref, w2_ref, b2_ref, os_ref):
    i = pl.program_id(0)
    e = ce_ref[i]
    s = pl.multiple_of(cs_ref[i], 8)
    x = hs_ref[pl.ds(s, T), :]
    h = jnp.dot(x, w1_ref[0, 0], preferred_element_type=jnp.float32)
    h = h + b1_ref[0, pl.ds(e, 1), :]
    h = 0.5 * h * (1.0 + lax.erf(h * 0.7071067811865476))
    y = jnp.dot(h, w2_ref[0, 0], preferred_element_type=jnp.float32)
    y = y + b2_ref[0, pl.ds(e, 1), :]
    os_ref[pl.ds(s, T), :] = y + x


def _make_ffn(l):
    return pl.pallas_call(
        _ffn_body,
        grid_spec=pltpu.PrefetchScalarGridSpec(
            num_scalar_prefetch=2,
            grid=(NCH,),
            in_specs=[
                pl.BlockSpec((NPAD, DIM), lambda i, ce, cs: (0, 0)),
                pl.BlockSpec((1, 1, DIM, FF), lambda i, ce, cs: (l, ce[i], 0, 0)),
                pl.BlockSpec((1, E, FF), lambda i, ce, cs: (l, 0, 0)),
                pl.BlockSpec((1, 1, FF, DIM), lambda i, ce, cs: (l, ce[i], 0, 0)),
                pl.BlockSpec((1, E, DIM), lambda i, ce, cs: (l, 0, 0)),
            ],
            out_specs=pl.BlockSpec((NPAD, DIM), lambda i, ce, cs: (0, 0)),
        ),
        out_shape=jax.ShapeDtypeStruct((NPAD, DIM), jnp.float32),
    )


def kernel(x, pos_emb, route_W, route_b, W1, b1, W2, b2, return_logits):
    x2 = x.reshape(SEQ, DIM)
    p2 = pos_emb.reshape(-1, DIM)[:SEQ]
    scatter_np = _make_sc_scatter(DIM)
    gather_sq = _make_sc_gather(SEQ, DIM)

    h0, slot0, ce0, cs0 = _route0(x2, p2, route_W[0], route_b[0].reshape(1, E))
    hs0 = scatter_np(h0, slot0.reshape(NW, SEQ // NW))
    os0 = _make_ffn(0)(ce0.reshape(NCHP), cs0.reshape(NCHP), hs0,
                       W1, b1, W2, b2)
    h1 = gather_sq(os0, slot0.reshape(SEQ))

    slot1, ce1, cs1 = _route1(h1, route_W[1], route_b[1].reshape(1, E))
    hs1 = scatter_np(h1, slot1.reshape(NW, SEQ // NW))
    os1 = _make_ffn(1)(ce1.reshape(NCHP), cs1.reshape(NCHP), hs1,
                       W1, b1, W2, b2)
    h2 = gather_sq(os1, slot1.reshape(SEQ))
    return h2.reshape(1, SEQ, DIM)
